# baseline (speedup 1.0000x reference)
import functools
import math

import jax
import jax.numpy as jnp
from jax.experimental import pallas as pl

LABEL_SMOOTHING = 0.1
IGNORE_INDEX = -100
ROW_BLOCK = 128
COL_CHUNK = 3200


def _loss_body(x_ref, t_ref, o_ref, *, V, B, RB, CB):
    s = LABEL_SMOOTHING / (V - 2)
    c = 1.0 - LABEL_SMOOTHING
    Z = V + IGNORE_INDEX
    kc = (V - 2) * s * math.log(s) + c * math.log(c)
    s_log_s = s * math.log(s)
    nch = V // CB

    i = pl.program_id(0)
    t = t_ref[0]
    cols0 = jax.lax.broadcasted_iota(jnp.int32, (RB, CB), 1)

    mx = jnp.full((RB, 1), -jnp.inf, dtype=jnp.float32)
    rs = jnp.zeros((RB, 1), dtype=jnp.float32)
    for j in range(nch):
        ch = x_ref[:, j * CB:(j + 1) * CB]
        mx = jnp.maximum(mx, jnp.max(ch, axis=1, keepdims=True))
        rs = rs + jnp.sum(ch, axis=1, keepdims=True)

    se = jnp.zeros((RB, 1), dtype=jnp.float32)
    xt = jnp.zeros((RB, 1), dtype=jnp.float32)
    for j in range(nch):
        ch = x_ref[:, j * CB:(j + 1) * CB]
        se = se + jnp.sum(jnp.exp(ch - mx), axis=1, keepdims=True)
        xt = xt + jnp.sum(jnp.where(cols0 == t - j * CB, ch, 0.0),
                          axis=1, keepdims=True)

    a = mx + jnp.log(se)
    xz = x_ref[:, Z:Z + 1]
    lp_t = xt - a
    lp_z = xz - a
    ssum = rs - V * a
    loss = kc - s * ssum + (s - c) * lp_t + s * lp_z
    loss = loss + jnp.where(t == Z, s_log_s - s * lp_z, 0.0)
    loss = jnp.where(t == IGNORE_INDEX, 0.0, loss)
    part = jnp.sum(loss, keepdims=True) * (1.0 / B)

    @pl.when(i == 0)
    def _():
        o_ref[...] = jnp.zeros_like(o_ref)

    o_ref[...] += part


def kernel(output, target, one_hot):
    B, V = output.shape
    RB = ROW_BLOCK
    G = B // RB
    t3 = target.reshape(G, RB, 1)
    out = pl.pallas_call(
        functools.partial(_loss_body, V=V, B=B, RB=RB, CB=COL_CHUNK),
        grid=(G,),
        in_specs=[
            pl.BlockSpec((RB, V), lambda i: (i, 0)),
            pl.BlockSpec((1, RB, 1), lambda i: (i, 0, 0)),
        ],
        out_specs=pl.BlockSpec((1, 1), lambda i: (0, 0)),
        out_shape=jax.ShapeDtypeStruct((1, 1), jnp.float32),
    )(output, t3)
    return out[0, 0]
